# baseline (speedup 1.0000x reference)
import jax
import jax.numpy as jnp
from jax import lax
from jax.experimental import pallas as pl
from jax.experimental.pallas import tpu as pltpu
from jax.experimental.pallas import tpu_sc as plsc

F32 = jnp.float32
I32 = jnp.int32

NC = 2
NS = 16
C = 128
NB = 4
CSET = 2 * C
EPAD = CSET * NS * NB
WC = 200

_mesh = plsc.VectorSubcoreMesh(core_axis_name="c", subcore_axis_name="s")
_sc_params = pltpu.CompilerParams(
    needs_layout_passes=False, use_tc_tiling_on_sc=False)


def _cdiv(a, b):
    return -(-a // b)


def _e16(e):
    return jnp.full((16,), e, I32)


def _spmm_acc(n_out, alpha, set_counts):
    ACC = _cdiv(n_out, C) * C
    nops = len(set_counts)

    def body(*refs):
        ins = refs[: 2 * nops]
        out_h = refs[2 * nops]
        sc = refs[2 * nops + 1:]
        mbuf = sc[0:NB]
        gbuf = sc[NB:2 * NB]
        obuf = sc[2 * NB]
        acc = sc[2 * NB + 1]
        sem_m = sc[2 * NB + 2:2 * NB + 2 + NB]
        sem_g = sc[2 * NB + 2 + NB:2 * NB + 2 + 2 * NB]
        sem_s = sc[2 * NB + 2 + 2 * NB:2 * NB + 2 + 3 * NB]

        cid = lax.axis_index("c")
        sid = lax.axis_index("s")

        zero16 = jnp.zeros((16,), F32)

        @pl.loop(0, C)
        def _zg(r):
            gbuf[0][r, pl.ds(0, 16)] = zero16

        n_zc = ACC // C

        @pl.loop(0, _cdiv(n_zc, NS))
        def _za(j):
            z = sid + j * NS

            @pl.when(z < n_zc)
            def _():
                pltpu.sync_copy(gbuf[0].at[pl.ds(0, C)],
                                acc.at[pl.ds(z * C, C)])

        plsc.subcore_barrier()

        for oi in range(nops):
            meta_h, y2_h = ins[2 * oi: 2 * oi + 2]
            n_set = set_counts[oi]
            M = n_set // NS

            def issue_meta(k, st, meta_h=meta_h):
                pltpu.async_copy(meta_h.at[st], mbuf[k], sem_m[k])

            def wait_meta(k, meta_h=meta_h):
                pltpu.make_async_copy(meta_h.at[0], mbuf[k], sem_m[k]).wait()

            def xform_gather(k, y2_h=y2_h):
                for h in range(2):
                    @pl.loop(0, C // 16)
                    def _(g, h=h):
                        cv = mbuf[k][2 + h, pl.ds(g * 16, 16)]
                        mbuf[k][2 + h, pl.ds(g * 16, 16)] = cv + cid

                for h in range(2):
                    pltpu.async_copy(
                        y2_h.at[mbuf[k].at[2 + h]],
                        gbuf[k].at[pl.ds(h * C, C)], sem_g[k])

            def wait_gather(k, y2_h=y2_h):
                for h in range(2):
                    pltpu.make_async_copy(
                        y2_h.at[pl.ds(0, C)],
                        gbuf[k].at[pl.ds(h * C, C)], sem_g[k]).wait()

            def scale(k):
                for h in range(2):
                    @pl.loop(0, C // 16)
                    def _(g, h=h):
                        vv = lax.bitcast_convert_type(
                            mbuf[k][4 + h, pl.ds(g * 16, 16)], F32)
                        for e in range(16):
                            sv = vv.at[_e16(e)].get(mode="promise_in_bounds")
                            r = h * C + g * 16 + e
                            gbuf[k][r, pl.ds(0, 16)] = (
                                gbuf[k][r, pl.ds(0, 16)] * sv)

            def issue_scatter(k):
                for h in range(2):
                    pltpu.async_copy(
                        gbuf[k].at[pl.ds(h * C, C)],
                        acc.at[mbuf[k].at[h]], sem_s[k], add=True)

            def wait_scatter(k):
                for h in range(2):
                    pltpu.make_async_copy(
                        gbuf[k].at[pl.ds(h * C, C)],
                        acc.at[pl.ds(0, C)], sem_s[k]).wait()

            def set_of(j):
                return sid + j * NS

            issue_meta(0, set_of(0))
            issue_meta(1, set_of(1))
            wait_meta(0)
            xform_gather(0)

            @pl.loop(0, M, step=NB)
            def _main(j0):
                for t in range(NB):
                    k = t
                    j = j0 + t
                    wait_gather(k)

                    @pl.when(j >= 2)
                    def _():
                        wait_scatter((t + 2) % NB)

                    @pl.when(j + 2 < M)
                    def _():
                        issue_meta((t + 2) % NB, set_of(j + 2))

                    @pl.when(j + 1 < M)
                    def _():
                        wait_meta((t + 1) % NB)
                        xform_gather((t + 1) % NB)

                    scale(k)
                    issue_scatter(k)

            wait_scatter((M - 2) % NB)
            wait_scatter((M - 1) % NB)

        plsc.subcore_barrier()

        n_wc = n_out // WC
        a = jnp.float32(alpha)

        @pl.loop(0, _cdiv(n_wc, NS))
        def _wb(j):
            w = sid + j * NS

            @pl.when(w < n_wc)
            def _():
                pltpu.sync_copy(acc.at[pl.ds(w * WC, WC)], obuf)

                @pl.loop(0, WC)
                def _r(r):
                    lo = obuf[r, pl.ds(0, 16)]
                    obuf[r, pl.ds(0, 16)] = jnp.maximum(lo, 0.0) * a

                pltpu.sync_copy(obuf, out_h.at[cid, pl.ds(w * WC, WC)])

    return pl.kernel(
        body,
        out_type=jax.ShapeDtypeStruct((NC, n_out, 16), F32),
        mesh=_mesh,
        compiler_params=_sc_params,
        scratch_types=(
            [pltpu.VMEM((6, C), I32) for _ in range(NB)]
            + [pltpu.VMEM((CSET, 16), F32) for _ in range(NB)]
            + [pltpu.VMEM((WC, 16), F32)]
            + [pltpu.VMEM_SHARED((ACC, 16), F32)]
            + [pltpu.SemaphoreType.DMA for _ in range(3 * NB)]
        ),
    )


def _spmm_full(n_out, set_counts):
    ACC = _cdiv(n_out, C) * C
    nops = len(set_counts)
    NW = NC * NS
    NBF = 3
    CF = 2 * C

    def body(*refs):
        ins = refs[: 2 * nops]
        out_h = refs[2 * nops]
        sc = refs[2 * nops + 1:]
        mbuf = sc[0:NBF]
        gbuf = sc[NBF:2 * NBF]
        acc = sc[2 * NBF]
        sem_m = sc[2 * NBF + 1:2 * NBF + 1 + NBF]
        sem_g = sc[2 * NBF + 1 + NBF:2 * NBF + 1 + 2 * NBF]
        sem_s = sc[2 * NBF + 1 + 2 * NBF:2 * NBF + 1 + 3 * NBF]

        cid = lax.axis_index("c")
        sid = lax.axis_index("s")
        wid = sid * NC + cid

        zero16 = jnp.zeros((16,), F32)

        @pl.loop(0, C)
        def _zg(r):
            gbuf[0][r, pl.ds(0, 16)] = zero16
            gbuf[0][r, pl.ds(16, 16)] = zero16

        n_zc = ACC // C

        @pl.loop(0, _cdiv(n_zc, NS))
        def _za(j):
            z = sid + j * NS

            @pl.when(z < n_zc)
            def _():
                pltpu.sync_copy(gbuf[0].at[pl.ds(0, C)],
                                acc.at[pl.ds(z * C, C)])

        plsc.subcore_barrier()

        for oi in range(nops):
            meta_h, y_h = ins[2 * oi: 2 * oi + 2]
            n_set = set_counts[oi]
            M = n_set // NW

            def issue_meta(k, st, meta_h=meta_h):
                pltpu.async_copy(meta_h.at[st], mbuf[k], sem_m[k])

            def wait_meta(k, meta_h=meta_h):
                pltpu.make_async_copy(meta_h.at[0], mbuf[k], sem_m[k]).wait()

            def issue_gather(k, y_h=y_h):
                for h in range(2):
                    pltpu.async_copy(
                        y_h.at[mbuf[k].at[2 + h]],
                        gbuf[k].at[pl.ds(h * C, C)], sem_g[k])

            def wait_gather(k, y_h=y_h):
                for h in range(2):
                    pltpu.make_async_copy(
                        y_h.at[pl.ds(0, C)],
                        gbuf[k].at[pl.ds(h * C, C)], sem_g[k]).wait()

            def scale(k):
                for h in range(2):
                    @pl.loop(0, C // 16)
                    def _(g, h=h):
                        vv = lax.bitcast_convert_type(
                            mbuf[k][4 + h, pl.ds(g * 16, 16)], F32)
                        for e in range(16):
                            sv = vv.at[_e16(e)].get(mode="promise_in_bounds")
                            r = h * C + g * 16 + e
                            gbuf[k][r, pl.ds(0, 16)] = (
                                gbuf[k][r, pl.ds(0, 16)] * sv)
                            gbuf[k][r, pl.ds(16, 16)] = (
                                gbuf[k][r, pl.ds(16, 16)] * sv)

            def issue_scatter(k):
                for h in range(2):
                    pltpu.async_copy(
                        gbuf[k].at[pl.ds(h * C, C)],
                        acc.at[mbuf[k].at[h]], sem_s[k], add=True)

            def wait_scatter(k):
                for h in range(2):
                    pltpu.make_async_copy(
                        gbuf[k].at[pl.ds(h * C, C)],
                        acc.at[pl.ds(0, C)], sem_s[k]).wait()

            def set_of(j):
                return wid + j * NW

            issue_meta(0, set_of(0))
            issue_meta(1, set_of(1))
            wait_meta(0)
            issue_gather(0)

            @pl.loop(0, M, step=NBF)
            def _main(j0):
                for t in range(NBF):
                    k = t
                    j = j0 + t
                    wait_gather(k)

                    @pl.when(j >= 1)
                    def _():
                        wait_scatter((t + 2) % NBF)

                    @pl.when(j + 2 < M)
                    def _():
                        issue_meta((t + 2) % NBF, set_of(j + 2))

                    @pl.when(j + 1 < M)
                    def _():
                        wait_meta((t + 1) % NBF)
                        issue_gather((t + 1) % NBF)

                    scale(k)
                    issue_scatter(k)

            wait_scatter((M - 1) % NBF)

        plsc.subcore_barrier()

        n_wc = _cdiv(n_out, C)

        @pl.loop(0, _cdiv(n_wc, NS))
        def _wb(j):
            w = sid + j * NS

            @pl.when(w < n_wc)
            def _():
                base = jnp.minimum(w * C, n_out - C)
                pltpu.sync_copy(acc.at[pl.ds(base, C)],
                                out_h.at[cid, pl.ds(base, C)])

    return pl.kernel(
        body,
        out_type=jax.ShapeDtypeStruct((NC, n_out, 32), F32),
        mesh=_mesh,
        compiler_params=_sc_params,
        scratch_types=(
            [pltpu.VMEM((6, C), I32) for _ in range(3)]
            + [pltpu.VMEM((2 * C, 32), F32) for _ in range(3)]
            + [pltpu.VMEM_SHARED((ACC, 32), F32)]
            + [pltpu.SemaphoreType.DMA for _ in range(9)]
        ),
    )


def _pool(n_rows):
    ACC = 128

    def body(x_h, b_h, sums_h, cnts_h, idxv, gbuf, onesb, obuf, acc_s, acc_c):
        cid = lax.axis_index("c")
        sid = lax.axis_index("s")

        zero16 = jnp.zeros((16,), F32)
        one16 = jnp.ones((16,), F32)

        @pl.loop(0, C)
        def _init(r):
            gbuf[r, pl.ds(0, 16)] = zero16
            onesb[r, pl.ds(0, 16)] = one16

        @pl.when(sid == 0)
        def _():
            pltpu.sync_copy(gbuf, acc_s)
            pltpu.sync_copy(gbuf, acc_c)

        plsc.subcore_barrier()

        lane = lax.iota(I32, 16)
        n_ch = _cdiv(n_rows, C)

        @pl.loop(0, _cdiv(n_ch, NS))
        def _rows(j):
            ch = sid + j * NS

            @pl.when(ch < n_ch)
            def _():
                base0 = ch * C
                base = jnp.minimum(base0, n_rows - C)
                lane_lo = base0 - base
                pltpu.sync_copy(x_h.at[cid, pl.ds(base, C)], gbuf)
                pltpu.sync_copy(b_h.at[pl.ds(base, C)], idxv)
                for g in range(C // 16):
                    bv = idxv[pl.ds(g * 16, 16)]
                    ok = (bv >= 0) & (bv < 64) & ((lane + g * 16) >= lane_lo)
                    idxv[pl.ds(g * 16, 16)] = jnp.where(ok, bv, 64)
                pltpu.sync_copy(gbuf, acc_s.at[idxv], add=True)
                pltpu.sync_copy(onesb, acc_c.at[idxv], add=True)

        plsc.subcore_barrier()

        @pl.when(sid == 0)
        def _():
            pltpu.sync_copy(acc_s.at[pl.ds(0, 64)], obuf)
            pltpu.sync_copy(obuf, sums_h.at[cid])
            pltpu.sync_copy(acc_c.at[pl.ds(0, 64)], obuf)
            pltpu.sync_copy(obuf, cnts_h.at[cid])

    return pl.kernel(
        body,
        out_type=(jax.ShapeDtypeStruct((NC, 64, 16), F32),
                  jax.ShapeDtypeStruct((NC, 64, 16), F32)),
        mesh=_mesh,
        compiler_params=_sc_params,
        scratch_types=[
            pltpu.VMEM((C,), I32),
            pltpu.VMEM((C, 16), F32),
            pltpu.VMEM((C, 16), F32),
            pltpu.VMEM((64, 16), F32),
            pltpu.VMEM_SHARED((ACC, 16), F32),
            pltpu.VMEM_SHARED((ACC, 16), F32),
        ],
    )


def _pool_pairs(n_rows, alpha):
    ACC = 128

    def body(x_h, b_h, sums_h, cnts_h, idxv, gbuf, gbuf2, onesb, obuf,
             acc_s, acc_c):
        cid = lax.axis_index("c")
        sid = lax.axis_index("s")
        wid = sid * NC + cid

        zero16 = jnp.zeros((16,), F32)
        one16 = jnp.ones((16,), F32)
        a = jnp.float32(alpha)

        @pl.loop(0, C)
        def _init(r):
            gbuf[r, pl.ds(0, 16)] = zero16
            gbuf[r, pl.ds(16, 16)] = zero16
            onesb[r, pl.ds(0, 16)] = one16
            onesb[r, pl.ds(16, 16)] = one16

        @pl.when(sid == 0)
        def _():
            pltpu.sync_copy(gbuf, acc_s)
            pltpu.sync_copy(gbuf, acc_c)

        plsc.subcore_barrier()

        lane = lax.iota(I32, 16)
        n_ch = _cdiv(n_rows, C)

        @pl.loop(0, _cdiv(n_ch, NC * NS))
        def _rows(j):
            ch = wid + j * NC * NS

            @pl.when(ch < n_ch)
            def _():
                base0 = ch * C
                base = jnp.minimum(base0, n_rows - C)
                lane_lo = base0 - base
                pltpu.sync_copy(x_h.at[0, pl.ds(base, C)], gbuf)
                pltpu.sync_copy(x_h.at[1, pl.ds(base, C)], gbuf2)
                pltpu.sync_copy(b_h.at[pl.ds(base, C)], idxv)

                @pl.loop(0, C)
                def _m(r):
                    lo = gbuf[r, pl.ds(0, 16)] + gbuf2[r, pl.ds(0, 16)]
                    gbuf[r, pl.ds(0, 16)] = jnp.maximum(lo, 0.0) * a
                    hi = gbuf[r, pl.ds(16, 16)] + gbuf2[r, pl.ds(16, 16)]
                    gbuf[r, pl.ds(16, 16)] = jnp.maximum(hi, 0.0) * a

                for g in range(C // 16):
                    bv = idxv[pl.ds(g * 16, 16)]
                    ok = (bv >= 0) & (bv < 64) & ((lane + g * 16) >= lane_lo)
                    idxv[pl.ds(g * 16, 16)] = jnp.where(ok, bv, 64)
                pltpu.sync_copy(gbuf, acc_s.at[idxv], add=True)
                pltpu.sync_copy(onesb, acc_c.at[idxv], add=True)

        plsc.subcore_barrier()

        @pl.when(sid == 0)
        def _():
            pltpu.sync_copy(acc_s.at[pl.ds(0, 64)], obuf)
            pltpu.sync_copy(obuf, sums_h.at[cid])
            pltpu.sync_copy(acc_c.at[pl.ds(0, 64)], obuf)
            pltpu.sync_copy(obuf, cnts_h.at[cid])

    return pl.kernel(
        body,
        out_type=(jax.ShapeDtypeStruct((NC, 64, 32), F32),
                  jax.ShapeDtypeStruct((NC, 64, 32), F32)),
        mesh=_mesh,
        compiler_params=_sc_params,
        scratch_types=[
            pltpu.VMEM((C,), I32),
            pltpu.VMEM((C, 32), F32),
            pltpu.VMEM((C, 32), F32),
            pltpu.VMEM((C, 32), F32),
            pltpu.VMEM((64, 32), F32),
            pltpu.VMEM_SHARED((ACC, 32), F32),
            pltpu.VMEM_SHARED((ACC, 32), F32),
        ],
    )


def _dense(X, Ws, bs, block_rows=1000, merge_alpha=None):
    split = X.ndim == 3 and X.shape[2] == 16
    pairs = X.ndim == 3 and X.shape[2] == 32
    n = X.shape[1] if (split or pairs) else X.shape[0]
    k = 32 if (split or pairs) else X.shape[1]
    ny = len(Ws)
    W = jnp.stack(Ws)
    b = jnp.stack(bs).reshape(ny, 1, 32)

    def body(x_ref, w_ref, b_ref, *outs):
        if split:
            x = jnp.concatenate([x_ref[0], x_ref[1]], axis=1)
        elif pairs:
            x = jnp.maximum(x_ref[0] + x_ref[1], 0.0) * merge_alpha
        else:
            x = x_ref[...]
        for i in range(ny):
            outs[i][...] = (
                jnp.dot(x, w_ref[i], preferred_element_type=F32) + b_ref[i]
            )

    if split:
        x_spec = pl.BlockSpec((2, block_rows, 16), lambda i: (0, i, 0))
    elif pairs:
        x_spec = pl.BlockSpec((2, block_rows, 32), lambda i: (0, i, 0))
    else:
        x_spec = pl.BlockSpec((block_rows, k), lambda i: (i, 0))
    return pl.pallas_call(
        body,
        grid=(n // block_rows,),
        in_specs=[
            x_spec,
            pl.BlockSpec((ny, k, 32), lambda i: (0, 0, 0)),
            pl.BlockSpec((ny, 1, 32), lambda i: (0, 0, 0)),
        ],
        out_specs=[pl.BlockSpec((block_rows, 32), lambda i: (i, 0))] * ny,
        out_shape=[jax.ShapeDtypeStruct((n, 32), F32)] * ny,
    )(X, W, b)


def _final(s0, c0, s1, c1, s2, c2, Wout, bout):
    def body(s0r, c0r, s1r, c1r, s2r, c2r, wr, br, outr):
        def pooled_planes(sr, cr):
            return jnp.concatenate(
                [sr[0] / jnp.maximum(cr[0], 1.0),
                 sr[1] / jnp.maximum(cr[1], 1.0)], axis=1)

        def pooled_pairs(sr, cr):
            return (sr[0] + sr[1]) / jnp.maximum(cr[0] + cr[1], 1.0)

        cat = jnp.concatenate(
            [pooled_pairs(s0r, c0r), pooled_planes(s1r, c1r),
             pooled_pairs(s2r, c2r)], axis=1)
        z = jnp.dot(cat, wr[...], preferred_element_type=F32) + br[...]
        z = z - jnp.max(z, axis=1, keepdims=True)
        ez = jnp.exp(z)
        outr[...] = ez / jnp.sum(ez, axis=1, keepdims=True)

    return pl.pallas_call(
        body,
        out_shape=jax.ShapeDtypeStruct((64, 32), F32),
    )(s0, c0, s1, c1, s2, c2, Wout, bout.reshape(1, 32))


def _pad_w(W, b):
    kin, kout = W.shape[1], W.shape[2]
    kin_pad = 128 if kin == 128 else 32
    Wp = jnp.zeros((7, kin_pad, 32), F32).at[:, :kin, :kout].set(W)
    bp = jnp.zeros((7, 32), F32).at[:, :kout].set(b)
    return Wp, bp


def _prep_op(rows, cols, vals, full=False):
    e = rows.shape[0]
    unit = CSET * NC * NS * 3 if full else EPAD
    ep = _cdiv(e, unit) * unit
    pad = (0, ep - e)
    r = jnp.pad(rows.astype(I32), pad)
    c = jnp.pad(cols.astype(I32), pad)
    if not full:
        c = c * 2
    v = lax.bitcast_convert_type(jnp.pad(vals.astype(F32), pad), I32)
    ns = ep // CSET
    return jnp.concatenate(
        [r.reshape(ns, 2, C), c.reshape(ns, 2, C), v.reshape(ns, 2, C)],
        axis=1)


def kernel(X0, X1, X2,
           L0_rows, L0_cols, L0_vals,
           L1_rows, L1_cols, L1_vals,
           L2_rows, L2_cols, L2_vals,
           B2D3_rows, B2D3_cols, B2D3_vals,
           D2B1TD1inv_rows, D2B1TD1inv_cols, D2B1TD1inv_vals,
           D1invB1_rows, D1invB1_cols, D1invB1_vals,
           B2TD2inv_rows, B2TD2inv_cols, B2TD2inv_vals,
           batch0, batch1, batch2,
           W1, b1, W2, b2, W3, b3, Wout, bout):
    ii = lambda x: x.astype(I32)
    ff = lambda x: x.astype(F32)

    ops = {
        "L0": _prep_op(L0_rows, L0_cols, L0_vals, full=True),
        "L1": _prep_op(L1_rows, L1_cols, L1_vals),
        "L2": _prep_op(L2_rows, L2_cols, L2_vals, full=True),
        "B2D3": _prep_op(B2D3_rows, B2D3_cols, B2D3_vals),
        "D2B1TD1inv": _prep_op(D2B1TD1inv_rows, D2B1TD1inv_cols,
                               D2B1TD1inv_vals),
        "D1invB1": _prep_op(D1invB1_rows, D1invB1_cols, D1invB1_vals,
                            full=True),
        "B2TD2inv": _prep_op(B2TD2inv_rows, B2TD2inv_cols, B2TD2inv_vals,
                             full=True),
    }

    def layer(x0, x1, x2, W, b):
        Wp, bp = _pad_w(W, b)
        a02 = None if x0.ndim == 2 else 0.5
        y_n2n, y_n2e = _dense(x0, [Wp[0], Wp[1]], [bp[0], bp[1]],
                              merge_alpha=a02)
        y_e2e, y_e2n, y_e2t = _dense(x1, [Wp[2], Wp[3], Wp[4]],
                                     [bp[2], bp[3], bp[4]])
        y_t2e, y_t2t = _dense(x2, [Wp[5], Wp[6]], [bp[5], bp[6]],
                              merge_alpha=a02)

        def run_split(n_out, alpha, pairs):
            counts = tuple(m.shape[0] for m, _ in pairs)
            args = []
            for m, y in pairs:
                args += [m, y.reshape(2 * y.shape[0], 16)]
            return _spmm_acc(n_out, alpha, counts)(*args)

        def run_full(n_out, pairs):
            counts = tuple(m.shape[0] for m, _ in pairs)
            args = []
            for m, y in pairs:
                args += [m, y]
            return _spmm_full(n_out, counts)(*args)

        o0 = run_full(50000, [(ops["L0"], y_n2n), (ops["D1invB1"], y_e2n)])
        o1 = run_split(100000, 1.0 / 3.0, [(ops["L1"], y_e2e),
                                           (ops["D2B1TD1inv"], y_n2e),
                                           (ops["B2D3"], y_t2e)])
        o2 = run_full(50000, [(ops["L2"], y_t2t), (ops["B2TD2inv"], y_e2t)])
        return o0, o1, o2

    x0, x1, x2 = ff(X0), ff(X1), ff(X2)
    x0, x1, x2 = layer(x0, x1, x2, W1, b1)
    x0, x1, x2 = layer(x0, x1, x2, W2, b2)
    x0, x1, x2 = layer(x0, x1, x2, W3, b3)

    s0, c0 = _pool_pairs(50000, 0.5)(x0, ii(batch0))
    s1, c1 = _pool(100000)(x1, ii(batch1))
    s2, c2 = _pool_pairs(50000, 0.5)(x2, ii(batch2))

    return _final(s0, c0, s1, c1, s2, c2, ff(Wout), ff(bout))


# speedup vs baseline: 1.0178x; 1.0178x over previous
import jax
import jax.numpy as jnp
from jax import lax
from jax.experimental import pallas as pl
from jax.experimental.pallas import tpu as pltpu
from jax.experimental.pallas import tpu_sc as plsc

F32 = jnp.float32
I32 = jnp.int32

NC = 2
NS = 16
C = 128
NB = 4
KS = 3
CSET = KS * C
EPAD = CSET * NS * NB
CF = 128
EPADF = CF * NC * NS * NB
WC = 100

_mesh = plsc.VectorSubcoreMesh(core_axis_name="c", subcore_axis_name="s")
_sc_params = pltpu.CompilerParams(
    needs_layout_passes=False, use_tc_tiling_on_sc=False)


def _cdiv(a, b):
    return -(-a // b)


def _e16(e):
    return jnp.full((16,), e, I32)


def _spmm_acc(n_out, alpha, set_counts):
    ACC = _cdiv(n_out, C) * C
    nops = len(set_counts)

    def body(*refs):
        ins = refs[: 2 * nops]
        out_h = refs[2 * nops]
        sc = refs[2 * nops + 1:]
        mbuf = sc[0:NB]
        gbuf = sc[NB:2 * NB]
        obuf = sc[2 * NB]
        acc = sc[2 * NB + 1]
        sem_m = sc[2 * NB + 2:2 * NB + 2 + NB]
        sem_g = sc[2 * NB + 2 + NB:2 * NB + 2 + 2 * NB]
        sem_s = sc[2 * NB + 2 + 2 * NB:2 * NB + 2 + 3 * NB]

        cid = lax.axis_index("c")
        sid = lax.axis_index("s")

        zero16 = jnp.zeros((16,), F32)

        @pl.loop(0, C)
        def _zg(r):
            gbuf[0][r, pl.ds(0, 16)] = zero16

        n_zc = ACC // C

        @pl.loop(0, _cdiv(n_zc, NS))
        def _za(j):
            z = sid + j * NS

            @pl.when(z < n_zc)
            def _():
                pltpu.sync_copy(gbuf[0].at[pl.ds(0, C)],
                                acc.at[pl.ds(z * C, C)])

        plsc.subcore_barrier()

        for oi in range(nops):
            meta_h, y2_h = ins[2 * oi: 2 * oi + 2]
            n_set = set_counts[oi]
            M = n_set // NS

            def issue_meta(k, st, meta_h=meta_h):
                pltpu.async_copy(meta_h.at[st], mbuf[k], sem_m[k])

            def wait_meta(k, meta_h=meta_h):
                pltpu.make_async_copy(meta_h.at[0], mbuf[k], sem_m[k]).wait()

            def xform_gather(k, y2_h=y2_h):
                for h in range(KS):
                    @pl.loop(0, C // 16)
                    def _(g, h=h):
                        cv = mbuf[k][KS + h, pl.ds(g * 16, 16)]
                        mbuf[k][KS + h, pl.ds(g * 16, 16)] = cv + cid

                for h in range(KS):
                    pltpu.async_copy(
                        y2_h.at[mbuf[k].at[KS + h]],
                        gbuf[k].at[pl.ds(h * C, C)], sem_g[k])

            def wait_gather(k, y2_h=y2_h):
                for h in range(KS):
                    pltpu.make_async_copy(
                        y2_h.at[pl.ds(0, C)],
                        gbuf[k].at[pl.ds(h * C, C)], sem_g[k]).wait()

            def scale(k):
                for h in range(KS):
                    @pl.loop(0, C // 16)
                    def _(g, h=h):
                        vv = lax.bitcast_convert_type(
                            mbuf[k][2 * KS + h, pl.ds(g * 16, 16)], F32)
                        for e in range(16):
                            sv = vv.at[_e16(e)].get(mode="promise_in_bounds")
                            r = h * C + g * 16 + e
                            gbuf[k][r, pl.ds(0, 16)] = (
                                gbuf[k][r, pl.ds(0, 16)] * sv)

            def issue_scatter(k):
                for h in range(KS):
                    pltpu.async_copy(
                        gbuf[k].at[pl.ds(h * C, C)],
                        acc.at[mbuf[k].at[h]], sem_s[k], add=True)

            def wait_scatter(k):
                for h in range(KS):
                    pltpu.make_async_copy(
                        gbuf[k].at[pl.ds(h * C, C)],
                        acc.at[pl.ds(0, C)], sem_s[k]).wait()

            def set_of(j):
                return sid + j * NS

            issue_meta(0, set_of(0))
            issue_meta(1, set_of(1))
            wait_meta(0)
            xform_gather(0)

            @pl.loop(0, M, step=NB)
            def _main(j0):
                for t in range(NB):
                    k = t
                    j = j0 + t
                    wait_gather(k)

                    @pl.when(j >= 2)
                    def _():
                        wait_scatter((t + 2) % NB)

                    @pl.when(j + 2 < M)
                    def _():
                        issue_meta((t + 2) % NB, set_of(j + 2))

                    @pl.when(j + 1 < M)
                    def _():
                        wait_meta((t + 1) % NB)
                        xform_gather((t + 1) % NB)

                    scale(k)
                    issue_scatter(k)

            wait_scatter((M - 2) % NB)
            wait_scatter((M - 1) % NB)

        plsc.subcore_barrier()

        n_wc = n_out // WC
        a = jnp.float32(alpha)

        @pl.loop(0, _cdiv(n_wc, NS))
        def _wb(j):
            w = sid + j * NS

            @pl.when(w < n_wc)
            def _():
                pltpu.sync_copy(acc.at[pl.ds(w * WC, WC)], obuf)

                @pl.loop(0, WC)
                def _r(r):
                    lo = obuf[r, pl.ds(0, 16)]
                    obuf[r, pl.ds(0, 16)] = jnp.maximum(lo, 0.0) * a

                pltpu.sync_copy(obuf, out_h.at[cid, pl.ds(w * WC, WC)])

    return pl.kernel(
        body,
        out_type=jax.ShapeDtypeStruct((NC, n_out, 16), F32),
        mesh=_mesh,
        compiler_params=_sc_params,
        scratch_types=(
            [pltpu.VMEM((3 * KS, C), I32) for _ in range(NB)]
            + [pltpu.VMEM((CSET, 16), F32) for _ in range(NB)]
            + [pltpu.VMEM((WC, 16), F32)]
            + [pltpu.VMEM_SHARED((ACC, 16), F32)]
            + [pltpu.SemaphoreType.DMA for _ in range(3 * NB)]
        ),
    )


def _spmm_full(n_out, set_counts):
    ACC = _cdiv(n_out, C) * C
    nops = len(set_counts)
    NW = NC * NS

    def body(*refs):
        ins = refs[: 2 * nops]
        out_h = refs[2 * nops]
        sc = refs[2 * nops + 1:]
        mbuf = sc[0:NB]
        gbuf = sc[NB:2 * NB]
        acc = sc[2 * NB]
        sem_m = sc[2 * NB + 1:2 * NB + 1 + NB]
        sem_g = sc[2 * NB + 1 + NB:2 * NB + 1 + 2 * NB]
        sem_s = sc[2 * NB + 1 + 2 * NB:2 * NB + 1 + 3 * NB]

        cid = lax.axis_index("c")
        sid = lax.axis_index("s")
        wid = sid * NC + cid

        zero16 = jnp.zeros((16,), F32)

        @pl.loop(0, C)
        def _zg(r):
            gbuf[0][r, pl.ds(0, 16)] = zero16
            gbuf[0][r, pl.ds(16, 16)] = zero16

        n_zc = ACC // C

        @pl.loop(0, _cdiv(n_zc, NS))
        def _za(j):
            z = sid + j * NS

            @pl.when(z < n_zc)
            def _():
                pltpu.sync_copy(gbuf[0], acc.at[pl.ds(z * C, C)])

        plsc.subcore_barrier()

        for oi in range(nops):
            meta_h, y_h = ins[2 * oi: 2 * oi + 2]
            n_set = set_counts[oi]
            M = n_set // NW

            def issue_meta(k, st, meta_h=meta_h):
                pltpu.async_copy(meta_h.at[st], mbuf[k], sem_m[k])

            def wait_meta(k, meta_h=meta_h):
                pltpu.make_async_copy(meta_h.at[0], mbuf[k], sem_m[k]).wait()

            def issue_gather(k, y_h=y_h):
                pltpu.async_copy(
                    y_h.at[mbuf[k].at[1]], gbuf[k], sem_g[k])

            def wait_gather(k, y_h=y_h):
                pltpu.make_async_copy(
                    y_h.at[pl.ds(0, C)], gbuf[k], sem_g[k]).wait()

            def scale(k):
                @pl.loop(0, C // 16)
                def _(g):
                    vv = lax.bitcast_convert_type(
                        mbuf[k][2, pl.ds(g * 16, 16)], F32)
                    for e in range(16):
                        sv = vv.at[_e16(e)].get(mode="promise_in_bounds")
                        r = g * 16 + e
                        gbuf[k][r, pl.ds(0, 16)] = (
                            gbuf[k][r, pl.ds(0, 16)] * sv)
                        gbuf[k][r, pl.ds(16, 16)] = (
                            gbuf[k][r, pl.ds(16, 16)] * sv)

            def issue_scatter(k):
                pltpu.async_copy(
                    gbuf[k], acc.at[mbuf[k].at[0]], sem_s[k], add=True)

            def wait_scatter(k):
                pltpu.make_async_copy(
                    gbuf[k], acc.at[pl.ds(0, C)], sem_s[k]).wait()

            def set_of(j):
                return wid + j * NW

            issue_meta(0, set_of(0))
            issue_meta(1, set_of(1))
            wait_meta(0)
            issue_gather(0)

            @pl.loop(0, M, step=NB)
            def _main(j0):
                for t in range(NB):
                    k = t
                    j = j0 + t
                    wait_gather(k)

                    @pl.when(j >= 2)
                    def _():
                        wait_scatter((t + 2) % NB)

                    @pl.when(j + 2 < M)
                    def _():
                        issue_meta((t + 2) % NB, set_of(j + 2))

                    @pl.when(j + 1 < M)
                    def _():
                        wait_meta((t + 1) % NB)
                        issue_gather((t + 1) % NB)

                    scale(k)
                    issue_scatter(k)

            wait_scatter((M - 2) % NB)
            wait_scatter((M - 1) % NB)

        plsc.subcore_barrier()

        n_wc = _cdiv(n_out, C)

        @pl.loop(0, _cdiv(n_wc, NS))
        def _wb(j):
            w = sid + j * NS

            @pl.when(w < n_wc)
            def _():
                base = jnp.minimum(w * C, n_out - C)
                pltpu.sync_copy(acc.at[pl.ds(base, C)],
                                out_h.at[cid, pl.ds(base, C)])

    return pl.kernel(
        body,
        out_type=jax.ShapeDtypeStruct((NC, n_out, 32), F32),
        mesh=_mesh,
        compiler_params=_sc_params,
        scratch_types=(
            [pltpu.VMEM((3, C), I32) for _ in range(NB)]
            + [pltpu.VMEM((C, 32), F32) for _ in range(NB)]
            + [pltpu.VMEM_SHARED((ACC, 32), F32)]
            + [pltpu.SemaphoreType.DMA for _ in range(3 * NB)]
        ),
    )


def _pool(n_rows):
    ACC = 128

    def body(x_h, b_h, sums_h, cnts_h, idxv, gbuf, onesb, obuf, acc_s, acc_c):
        cid = lax.axis_index("c")
        sid = lax.axis_index("s")

        zero16 = jnp.zeros((16,), F32)
        one16 = jnp.ones((16,), F32)

        @pl.loop(0, C)
        def _init(r):
            gbuf[r, pl.ds(0, 16)] = zero16
            onesb[r, pl.ds(0, 16)] = one16

        @pl.when(sid == 0)
        def _():
            pltpu.sync_copy(gbuf, acc_s)
            pltpu.sync_copy(gbuf, acc_c)

        plsc.subcore_barrier()

        lane = lax.iota(I32, 16)
        n_ch = _cdiv(n_rows, C)

        @pl.loop(0, _cdiv(n_ch, NS))
        def _rows(j):
            ch = sid + j * NS

            @pl.when(ch < n_ch)
            def _():
                base0 = ch * C
                base = jnp.minimum(base0, n_rows - C)
                lane_lo = base0 - base
                pltpu.sync_copy(x_h.at[cid, pl.ds(base, C)], gbuf)
                pltpu.sync_copy(b_h.at[pl.ds(base, C)], idxv)
                for g in range(C // 16):
                    bv = idxv[pl.ds(g * 16, 16)]
                    ok = (bv >= 0) & (bv < 64) & ((lane + g * 16) >= lane_lo)
                    idxv[pl.ds(g * 16, 16)] = jnp.where(ok, bv, 64)
                pltpu.sync_copy(gbuf, acc_s.at[idxv], add=True)
                pltpu.sync_copy(onesb, acc_c.at[idxv], add=True)

        plsc.subcore_barrier()

        @pl.when(sid == 0)
        def _():
            pltpu.sync_copy(acc_s.at[pl.ds(0, 64)], obuf)
            pltpu.sync_copy(obuf, sums_h.at[cid])
            pltpu.sync_copy(acc_c.at[pl.ds(0, 64)], obuf)
            pltpu.sync_copy(obuf, cnts_h.at[cid])

    return pl.kernel(
        body,
        out_type=(jax.ShapeDtypeStruct((NC, 64, 16), F32),
                  jax.ShapeDtypeStruct((NC, 64, 16), F32)),
        mesh=_mesh,
        compiler_params=_sc_params,
        scratch_types=[
            pltpu.VMEM((C,), I32),
            pltpu.VMEM((C, 16), F32),
            pltpu.VMEM((C, 16), F32),
            pltpu.VMEM((64, 16), F32),
            pltpu.VMEM_SHARED((ACC, 16), F32),
            pltpu.VMEM_SHARED((ACC, 16), F32),
        ],
    )


def _pool_pairs(n_rows, alpha):
    ACC = 128

    def body(x_h, b_h, sums_h, cnts_h, idxv, gbuf, gbuf2, onesb, obuf,
             acc_s, acc_c):
        cid = lax.axis_index("c")
        sid = lax.axis_index("s")
        wid = sid * NC + cid

        zero16 = jnp.zeros((16,), F32)
        one16 = jnp.ones((16,), F32)
        a = jnp.float32(alpha)

        @pl.loop(0, C)
        def _init(r):
            gbuf[r, pl.ds(0, 16)] = zero16
            gbuf[r, pl.ds(16, 16)] = zero16
            onesb[r, pl.ds(0, 16)] = one16
            onesb[r, pl.ds(16, 16)] = one16

        @pl.when(sid == 0)
        def _():
            pltpu.sync_copy(gbuf, acc_s)
            pltpu.sync_copy(gbuf, acc_c)

        plsc.subcore_barrier()

        lane = lax.iota(I32, 16)
        n_ch = _cdiv(n_rows, C)

        @pl.loop(0, _cdiv(n_ch, NC * NS))
        def _rows(j):
            ch = wid + j * NC * NS

            @pl.when(ch < n_ch)
            def _():
                base0 = ch * C
                base = jnp.minimum(base0, n_rows - C)
                lane_lo = base0 - base
                pltpu.sync_copy(x_h.at[0, pl.ds(base, C)], gbuf)
                pltpu.sync_copy(x_h.at[1, pl.ds(base, C)], gbuf2)
                pltpu.sync_copy(b_h.at[pl.ds(base, C)], idxv)

                @pl.loop(0, C)
                def _m(r):
                    lo = gbuf[r, pl.ds(0, 16)] + gbuf2[r, pl.ds(0, 16)]
                    gbuf[r, pl.ds(0, 16)] = jnp.maximum(lo, 0.0) * a
                    hi = gbuf[r, pl.ds(16, 16)] + gbuf2[r, pl.ds(16, 16)]
                    gbuf[r, pl.ds(16, 16)] = jnp.maximum(hi, 0.0) * a

                for g in range(C // 16):
                    bv = idxv[pl.ds(g * 16, 16)]
                    ok = (bv >= 0) & (bv < 64) & ((lane + g * 16) >= lane_lo)
                    idxv[pl.ds(g * 16, 16)] = jnp.where(ok, bv, 64)
                pltpu.sync_copy(gbuf, acc_s.at[idxv], add=True)
                pltpu.sync_copy(onesb, acc_c.at[idxv], add=True)

        plsc.subcore_barrier()

        @pl.when(sid == 0)
        def _():
            pltpu.sync_copy(acc_s.at[pl.ds(0, 64)], obuf)
            pltpu.sync_copy(obuf, sums_h.at[cid])
            pltpu.sync_copy(acc_c.at[pl.ds(0, 64)], obuf)
            pltpu.sync_copy(obuf, cnts_h.at[cid])

    return pl.kernel(
        body,
        out_type=(jax.ShapeDtypeStruct((NC, 64, 32), F32),
                  jax.ShapeDtypeStruct((NC, 64, 32), F32)),
        mesh=_mesh,
        compiler_params=_sc_params,
        scratch_types=[
            pltpu.VMEM((C,), I32),
            pltpu.VMEM((C, 32), F32),
            pltpu.VMEM((C, 32), F32),
            pltpu.VMEM((C, 32), F32),
            pltpu.VMEM((64, 32), F32),
            pltpu.VMEM_SHARED((ACC, 32), F32),
            pltpu.VMEM_SHARED((ACC, 32), F32),
        ],
    )


def _dense(X, Ws, bs, block_rows=1000, merge_alpha=None):
    split = X.ndim == 3 and X.shape[2] == 16
    pairs = X.ndim == 3 and X.shape[2] == 32
    n = X.shape[1] if (split or pairs) else X.shape[0]
    k = 32 if (split or pairs) else X.shape[1]
    ny = len(Ws)
    W = jnp.stack(Ws)
    b = jnp.stack(bs).reshape(ny, 1, 32)

    def body(x_ref, w_ref, b_ref, *outs):
        if split:
            x = jnp.concatenate([x_ref[0], x_ref[1]], axis=1)
        elif pairs:
            x = jnp.maximum(x_ref[0] + x_ref[1], 0.0) * merge_alpha
        else:
            x = x_ref[...]
        for i in range(ny):
            outs[i][...] = (
                jnp.dot(x, w_ref[i], preferred_element_type=F32) + b_ref[i]
            )

    if split:
        x_spec = pl.BlockSpec((2, block_rows, 16), lambda i: (0, i, 0))
    elif pairs:
        x_spec = pl.BlockSpec((2, block_rows, 32), lambda i: (0, i, 0))
    else:
        x_spec = pl.BlockSpec((block_rows, k), lambda i: (i, 0))
    return pl.pallas_call(
        body,
        grid=(n // block_rows,),
        in_specs=[
            x_spec,
            pl.BlockSpec((ny, k, 32), lambda i: (0, 0, 0)),
            pl.BlockSpec((ny, 1, 32), lambda i: (0, 0, 0)),
        ],
        out_specs=[pl.BlockSpec((block_rows, 32), lambda i: (i, 0))] * ny,
        out_shape=[jax.ShapeDtypeStruct((n, 32), F32)] * ny,
    )(X, W, b)


def _final(s0, c0, s1, c1, s2, c2, Wout, bout):
    def body(s0r, c0r, s1r, c1r, s2r, c2r, wr, br, outr):
        def pooled_planes(sr, cr):
            return jnp.concatenate(
                [sr[0] / jnp.maximum(cr[0], 1.0),
                 sr[1] / jnp.maximum(cr[1], 1.0)], axis=1)

        def pooled_pairs(sr, cr):
            return (sr[0] + sr[1]) / jnp.maximum(cr[0] + cr[1], 1.0)

        cat = jnp.concatenate(
            [pooled_pairs(s0r, c0r), pooled_planes(s1r, c1r),
             pooled_pairs(s2r, c2r)], axis=1)
        z = jnp.dot(cat, wr[...], preferred_element_type=F32) + br[...]
        z = z - jnp.max(z, axis=1, keepdims=True)
        ez = jnp.exp(z)
        outr[...] = ez / jnp.sum(ez, axis=1, keepdims=True)

    return pl.pallas_call(
        body,
        out_shape=jax.ShapeDtypeStruct((64, 32), F32),
    )(s0, c0, s1, c1, s2, c2, Wout, bout.reshape(1, 32))


def _pad_w(W, b):
    kin, kout = W.shape[1], W.shape[2]
    kin_pad = 128 if kin == 128 else 32
    Wp = jnp.zeros((7, kin_pad, 32), F32).at[:, :kin, :kout].set(W)
    bp = jnp.zeros((7, 32), F32).at[:, :kout].set(b)
    return Wp, bp


def _prep_op(rows, cols, vals, full=False):
    e = rows.shape[0]
    unit = EPADF if full else EPAD
    ep = _cdiv(e, unit) * unit
    pad = (0, ep - e)
    r = jnp.pad(rows.astype(I32), pad)
    c = jnp.pad(cols.astype(I32), pad)
    v = lax.bitcast_convert_type(jnp.pad(vals.astype(F32), pad), I32)
    if full:
        ns = ep // CF
        return jnp.concatenate(
            [r.reshape(ns, 1, C), c.reshape(ns, 1, C), v.reshape(ns, 1, C)],
            axis=1)
    ns = ep // CSET
    return jnp.concatenate(
        [r.reshape(ns, KS, C), (c * 2).reshape(ns, KS, C),
         v.reshape(ns, KS, C)], axis=1)


def kernel(X0, X1, X2,
           L0_rows, L0_cols, L0_vals,
           L1_rows, L1_cols, L1_vals,
           L2_rows, L2_cols, L2_vals,
           B2D3_rows, B2D3_cols, B2D3_vals,
           D2B1TD1inv_rows, D2B1TD1inv_cols, D2B1TD1inv_vals,
           D1invB1_rows, D1invB1_cols, D1invB1_vals,
           B2TD2inv_rows, B2TD2inv_cols, B2TD2inv_vals,
           batch0, batch1, batch2,
           W1, b1, W2, b2, W3, b3, Wout, bout):
    ii = lambda x: x.astype(I32)
    ff = lambda x: x.astype(F32)

    ops = {
        "L0": _prep_op(L0_rows, L0_cols, L0_vals, full=True),
        "L1": _prep_op(L1_rows, L1_cols, L1_vals),
        "L2": _prep_op(L2_rows, L2_cols, L2_vals, full=True),
        "B2D3": _prep_op(B2D3_rows, B2D3_cols, B2D3_vals),
        "D2B1TD1inv": _prep_op(D2B1TD1inv_rows, D2B1TD1inv_cols,
                               D2B1TD1inv_vals),
        "D1invB1": _prep_op(D1invB1_rows, D1invB1_cols, D1invB1_vals,
                            full=True),
        "B2TD2inv": _prep_op(B2TD2inv_rows, B2TD2inv_cols, B2TD2inv_vals,
                             full=True),
    }

    def layer(x0, x1, x2, W, b):
        Wp, bp = _pad_w(W, b)
        a02 = None if x0.ndim == 2 else 0.5
        y_n2n, y_n2e = _dense(x0, [Wp[0], Wp[1]], [bp[0], bp[1]],
                              merge_alpha=a02)
        y_e2e, y_e2n, y_e2t = _dense(x1, [Wp[2], Wp[3], Wp[4]],
                                     [bp[2], bp[3], bp[4]])
        y_t2e, y_t2t = _dense(x2, [Wp[5], Wp[6]], [bp[5], bp[6]],
                              merge_alpha=a02)

        def run_split(n_out, alpha, pairs):
            counts = tuple(m.shape[0] for m, _ in pairs)
            args = []
            for m, y in pairs:
                args += [m, y.reshape(2 * y.shape[0], 16)]
            return _spmm_acc(n_out, alpha, counts)(*args)

        def run_full(n_out, pairs):
            counts = tuple(m.shape[0] for m, _ in pairs)
            args = []
            for m, y in pairs:
                args += [m, y]
            return _spmm_full(n_out, counts)(*args)

        o0 = run_full(50000, [(ops["L0"], y_n2n), (ops["D1invB1"], y_e2n)])
        o1 = run_split(100000, 1.0 / 3.0, [(ops["L1"], y_e2e),
                                           (ops["D2B1TD1inv"], y_n2e),
                                           (ops["B2D3"], y_t2e)])
        o2 = run_full(50000, [(ops["L2"], y_t2t), (ops["B2TD2inv"], y_e2t)])
        return o0, o1, o2

    x0, x1, x2 = ff(X0), ff(X1), ff(X2)
    x0, x1, x2 = layer(x0, x1, x2, W1, b1)
    x0, x1, x2 = layer(x0, x1, x2, W2, b2)
    x0, x1, x2 = layer(x0, x1, x2, W3, b3)

    s0, c0 = _pool_pairs(50000, 0.5)(x0, ii(batch0))
    s1, c1 = _pool(100000)(x1, ii(batch1))
    s2, c2 = _pool_pairs(50000, 0.5)(x2, ii(batch2))

    return _final(s0, c0, s1, c1, s2, c2, ff(Wout), ff(bout))


# speedup vs baseline: 1.1338x; 1.1140x over previous
import jax
import jax.numpy as jnp
from jax import lax
from jax.experimental import pallas as pl
from jax.experimental.pallas import tpu as pltpu
from jax.experimental.pallas import tpu_sc as plsc

F32 = jnp.float32
I32 = jnp.int32

NC = 2
NS = 16
C = 128
NB = 4
CSET = 2 * C
EPAD = CSET * NS * NB
WC = 200

_mesh = plsc.VectorSubcoreMesh(core_axis_name="c", subcore_axis_name="s")
_sc_params = pltpu.CompilerParams(
    needs_layout_passes=False, use_tc_tiling_on_sc=False)


def _cdiv(a, b):
    return -(-a // b)


def _e16(e):
    return jnp.full((16,), e, I32)


def _spmm_acc(n_out, alpha, set_counts):
    ACC = _cdiv(n_out, C) * C
    nops = len(set_counts)

    def body(*refs):
        ins = refs[: 2 * nops]
        out_h = refs[2 * nops]
        sc = refs[2 * nops + 1:]
        mbuf = sc[0:NB]
        gbuf = sc[NB:2 * NB]
        obuf = sc[2 * NB]
        acc = sc[2 * NB + 1]
        sem_m = sc[2 * NB + 2:2 * NB + 2 + NB]
        sem_g = sc[2 * NB + 2 + NB:2 * NB + 2 + 2 * NB]
        sem_s = sc[2 * NB + 2 + 2 * NB:2 * NB + 2 + 3 * NB]

        cid = lax.axis_index("c")
        sid = lax.axis_index("s")

        zero16 = jnp.zeros((16,), F32)

        @pl.loop(0, C)
        def _zg(r):
            gbuf[0][r, pl.ds(0, 16)] = zero16

        n_zc = ACC // C

        @pl.loop(0, _cdiv(n_zc, NS))
        def _za(j):
            z = sid + j * NS

            @pl.when(z < n_zc)
            def _():
                pltpu.sync_copy(gbuf[0].at[pl.ds(0, C)],
                                acc.at[pl.ds(z * C, C)])

        plsc.subcore_barrier()

        for oi in range(nops):
            meta_h, y2_h = ins[2 * oi: 2 * oi + 2]
            n_set = set_counts[oi]
            M = n_set // NS

            def issue_meta(k, st, meta_h=meta_h):
                pltpu.async_copy(meta_h.at[st], mbuf[k], sem_m[k])

            def wait_meta(k, meta_h=meta_h):
                pltpu.make_async_copy(meta_h.at[0], mbuf[k], sem_m[k]).wait()

            def xform_gather(k, y2_h=y2_h):
                for h in range(2):
                    @pl.loop(0, C // 16)
                    def _(g, h=h):
                        cv = mbuf[k][2 + h, pl.ds(g * 16, 16)]
                        mbuf[k][2 + h, pl.ds(g * 16, 16)] = cv + cid

                for h in range(2):
                    pltpu.async_copy(
                        y2_h.at[mbuf[k].at[2 + h]],
                        gbuf[k].at[pl.ds(h * C, C)], sem_g[k])

            def wait_gather(k, y2_h=y2_h):
                for h in range(2):
                    pltpu.make_async_copy(
                        y2_h.at[pl.ds(0, C)],
                        gbuf[k].at[pl.ds(h * C, C)], sem_g[k]).wait()

            def scale(k):
                for h in range(2):
                    @pl.loop(0, C // 16)
                    def _(g, h=h):
                        vv = lax.bitcast_convert_type(
                            mbuf[k][4 + h, pl.ds(g * 16, 16)], F32)
                        for e in range(16):
                            sv = vv.at[_e16(e)].get(mode="promise_in_bounds")
                            r = h * C + g * 16 + e
                            gbuf[k][r, pl.ds(0, 16)] = (
                                gbuf[k][r, pl.ds(0, 16)] * sv)

            def issue_scatter(k):
                for h in range(2):
                    pltpu.async_copy(
                        gbuf[k].at[pl.ds(h * C, C)],
                        acc.at[mbuf[k].at[h]], sem_s[k], add=True)

            def wait_scatter(k):
                for h in range(2):
                    pltpu.make_async_copy(
                        gbuf[k].at[pl.ds(h * C, C)],
                        acc.at[pl.ds(0, C)], sem_s[k]).wait()

            def set_of(j):
                return sid + j * NS

            issue_meta(0, set_of(0))
            issue_meta(1, set_of(1))
            wait_meta(0)
            xform_gather(0)

            @pl.loop(0, M, step=NB)
            def _main(j0):
                for t in range(NB):
                    k = t
                    j = j0 + t
                    wait_gather(k)

                    @pl.when(j >= 2)
                    def _():
                        wait_scatter((t + 2) % NB)

                    @pl.when(j + 2 < M)
                    def _():
                        issue_meta((t + 2) % NB, set_of(j + 2))

                    @pl.when(j + 1 < M)
                    def _():
                        wait_meta((t + 1) % NB)
                        xform_gather((t + 1) % NB)

                    scale(k)
                    issue_scatter(k)

            wait_scatter((M - 2) % NB)
            wait_scatter((M - 1) % NB)

        plsc.subcore_barrier()

        n_wc = n_out // WC
        a = jnp.float32(alpha)

        @pl.loop(0, _cdiv(n_wc, NS))
        def _wb(j):
            w = sid + j * NS

            @pl.when(w < n_wc)
            def _():
                pltpu.sync_copy(acc.at[pl.ds(w * WC, WC)], obuf)

                @pl.loop(0, WC)
                def _r(r):
                    lo = obuf[r, pl.ds(0, 16)]
                    obuf[r, pl.ds(0, 16)] = jnp.maximum(lo, 0.0) * a

                pltpu.sync_copy(obuf, out_h.at[cid, pl.ds(w * WC, WC)])

    return pl.kernel(
        body,
        out_type=jax.ShapeDtypeStruct((NC, n_out, 16), F32),
        mesh=_mesh,
        compiler_params=_sc_params,
        scratch_types=(
            [pltpu.VMEM((6, C), I32) for _ in range(NB)]
            + [pltpu.VMEM((CSET, 16), F32) for _ in range(NB)]
            + [pltpu.VMEM((WC, 16), F32)]
            + [pltpu.VMEM_SHARED((ACC, 16), F32)]
            + [pltpu.SemaphoreType.DMA for _ in range(3 * NB)]
        ),
    )


def _spmm_full(n_out, set_counts):
    ACC = _cdiv(n_out, C) * C
    nops = len(set_counts)
    NW = NC * NS

    def body(*refs):
        ins = refs[: 2 * nops]
        out_h = refs[2 * nops]
        sc = refs[2 * nops + 1:]
        mbuf = sc[0:NB]
        gbuf = sc[NB:2 * NB]
        acc = sc[2 * NB]
        sem_m = sc[2 * NB + 1:2 * NB + 1 + NB]
        sem_g = sc[2 * NB + 1 + NB:2 * NB + 1 + 2 * NB]
        sem_s = sc[2 * NB + 1 + 2 * NB:2 * NB + 1 + 3 * NB]

        cid = lax.axis_index("c")
        sid = lax.axis_index("s")
        wid = sid * NC + cid

        zero16 = jnp.zeros((16,), F32)

        @pl.loop(0, C)
        def _zg(r):
            gbuf[0][r, pl.ds(0, 16)] = zero16
            gbuf[0][r, pl.ds(16, 16)] = zero16

        n_zc = ACC // C

        @pl.loop(0, _cdiv(n_zc, NS))
        def _za(j):
            z = sid + j * NS

            @pl.when(z < n_zc)
            def _():
                pltpu.sync_copy(gbuf[0], acc.at[pl.ds(z * C, C)])

        plsc.subcore_barrier()

        for oi in range(nops):
            meta_h, y_h = ins[2 * oi: 2 * oi + 2]
            n_set = set_counts[oi]
            M = n_set // NW

            def issue_meta(k, st, meta_h=meta_h):
                pltpu.async_copy(meta_h.at[st], mbuf[k], sem_m[k])

            def wait_meta(k, meta_h=meta_h):
                pltpu.make_async_copy(meta_h.at[0], mbuf[k], sem_m[k]).wait()

            def issue_gather(k, y_h=y_h):
                pltpu.async_copy(
                    y_h.at[mbuf[k].at[1]], gbuf[k], sem_g[k])

            def wait_gather(k, y_h=y_h):
                pltpu.make_async_copy(
                    y_h.at[pl.ds(0, C)], gbuf[k], sem_g[k]).wait()

            def scale(k):
                @pl.loop(0, C // 16)
                def _(g):
                    vv = lax.bitcast_convert_type(
                        mbuf[k][2, pl.ds(g * 16, 16)], F32)
                    for e in range(16):
                        sv = vv.at[_e16(e)].get(mode="promise_in_bounds")
                        r = g * 16 + e
                        gbuf[k][r, pl.ds(0, 16)] = (
                            gbuf[k][r, pl.ds(0, 16)] * sv)
                        gbuf[k][r, pl.ds(16, 16)] = (
                            gbuf[k][r, pl.ds(16, 16)] * sv)

            def issue_scatter(k):
                pltpu.async_copy(
                    gbuf[k], acc.at[mbuf[k].at[0]], sem_s[k], add=True)

            def wait_scatter(k):
                pltpu.make_async_copy(
                    gbuf[k], acc.at[pl.ds(0, C)], sem_s[k]).wait()

            def set_of(j):
                return wid + j * NW

            issue_meta(0, set_of(0))
            issue_meta(1, set_of(1))
            wait_meta(0)
            issue_gather(0)

            @pl.loop(0, M, step=NB)
            def _main(j0):
                for t in range(NB):
                    k = t
                    j = j0 + t
                    wait_gather(k)

                    @pl.when(j >= 2)
                    def _():
                        wait_scatter((t + 2) % NB)

                    @pl.when(j + 2 < M)
                    def _():
                        issue_meta((t + 2) % NB, set_of(j + 2))

                    @pl.when(j + 1 < M)
                    def _():
                        wait_meta((t + 1) % NB)
                        issue_gather((t + 1) % NB)

                    scale(k)
                    issue_scatter(k)

            wait_scatter((M - 2) % NB)
            wait_scatter((M - 1) % NB)

        plsc.subcore_barrier()

        n_wc = _cdiv(n_out, C)

        @pl.loop(0, _cdiv(n_wc, NS))
        def _wb(j):
            w = sid + j * NS

            @pl.when(w < n_wc)
            def _():
                base = jnp.minimum(w * C, n_out - C)
                pltpu.sync_copy(acc.at[pl.ds(base, C)],
                                out_h.at[cid, pl.ds(base, C)])

    return pl.kernel(
        body,
        out_type=jax.ShapeDtypeStruct((NC, n_out, 32), F32),
        mesh=_mesh,
        compiler_params=_sc_params,
        scratch_types=(
            [pltpu.VMEM((3, C), I32) for _ in range(NB)]
            + [pltpu.VMEM((C, 32), F32) for _ in range(NB)]
            + [pltpu.VMEM_SHARED((ACC, 32), F32)]
            + [pltpu.SemaphoreType.DMA for _ in range(3 * NB)]
        ),
    )


def _pool(n_rows):
    ACC = 128

    def body(x_h, b_h, sums_h, cnts_h, idxv, gbuf, onesb, obuf, acc_s, acc_c):
        cid = lax.axis_index("c")
        sid = lax.axis_index("s")

        zero16 = jnp.zeros((16,), F32)
        one16 = jnp.ones((16,), F32)

        @pl.loop(0, C)
        def _init(r):
            gbuf[r, pl.ds(0, 16)] = zero16
            onesb[r, pl.ds(0, 16)] = one16

        @pl.when(sid == 0)
        def _():
            pltpu.sync_copy(gbuf, acc_s)
            pltpu.sync_copy(gbuf, acc_c)

        plsc.subcore_barrier()

        lane = lax.iota(I32, 16)
        n_ch = _cdiv(n_rows, C)

        @pl.loop(0, _cdiv(n_ch, NS))
        def _rows(j):
            ch = sid + j * NS

            @pl.when(ch < n_ch)
            def _():
                base0 = ch * C
                base = jnp.minimum(base0, n_rows - C)
                lane_lo = base0 - base
                pltpu.sync_copy(x_h.at[cid, pl.ds(base, C)], gbuf)
                pltpu.sync_copy(b_h.at[pl.ds(base, C)], idxv)
                for g in range(C // 16):
                    bv = idxv[pl.ds(g * 16, 16)]
                    ok = (bv >= 0) & (bv < 64) & ((lane + g * 16) >= lane_lo)
                    idxv[pl.ds(g * 16, 16)] = jnp.where(ok, bv, 64)
                pltpu.sync_copy(gbuf, acc_s.at[idxv], add=True)
                pltpu.sync_copy(onesb, acc_c.at[idxv], add=True)

        plsc.subcore_barrier()

        @pl.when(sid == 0)
        def _():
            pltpu.sync_copy(acc_s.at[pl.ds(0, 64)], obuf)
            pltpu.sync_copy(obuf, sums_h.at[cid])
            pltpu.sync_copy(acc_c.at[pl.ds(0, 64)], obuf)
            pltpu.sync_copy(obuf, cnts_h.at[cid])

    return pl.kernel(
        body,
        out_type=(jax.ShapeDtypeStruct((NC, 64, 16), F32),
                  jax.ShapeDtypeStruct((NC, 64, 16), F32)),
        mesh=_mesh,
        compiler_params=_sc_params,
        scratch_types=[
            pltpu.VMEM((C,), I32),
            pltpu.VMEM((C, 16), F32),
            pltpu.VMEM((C, 16), F32),
            pltpu.VMEM((64, 16), F32),
            pltpu.VMEM_SHARED((ACC, 16), F32),
            pltpu.VMEM_SHARED((ACC, 16), F32),
        ],
    )


def _pool_pairs(n_rows, alpha):
    ACC = 128

    def body(x_h, b_h, sums_h, cnts_h, idxv, gbuf, gbuf2, onesb, obuf,
             acc_s, acc_c):
        cid = lax.axis_index("c")
        sid = lax.axis_index("s")
        wid = sid * NC + cid

        zero16 = jnp.zeros((16,), F32)
        one16 = jnp.ones((16,), F32)
        a = jnp.float32(alpha)

        @pl.loop(0, C)
        def _init(r):
            gbuf[r, pl.ds(0, 16)] = zero16
            gbuf[r, pl.ds(16, 16)] = zero16
            onesb[r, pl.ds(0, 16)] = one16
            onesb[r, pl.ds(16, 16)] = one16

        @pl.when(sid == 0)
        def _():
            pltpu.sync_copy(gbuf, acc_s)
            pltpu.sync_copy(gbuf, acc_c)

        plsc.subcore_barrier()

        lane = lax.iota(I32, 16)
        n_ch = _cdiv(n_rows, C)

        @pl.loop(0, _cdiv(n_ch, NC * NS))
        def _rows(j):
            ch = wid + j * NC * NS

            @pl.when(ch < n_ch)
            def _():
                base0 = ch * C
                base = jnp.minimum(base0, n_rows - C)
                lane_lo = base0 - base
                pltpu.sync_copy(x_h.at[0, pl.ds(base, C)], gbuf)
                pltpu.sync_copy(x_h.at[1, pl.ds(base, C)], gbuf2)
                pltpu.sync_copy(b_h.at[pl.ds(base, C)], idxv)

                @pl.loop(0, C)
                def _m(r):
                    lo = gbuf[r, pl.ds(0, 16)] + gbuf2[r, pl.ds(0, 16)]
                    gbuf[r, pl.ds(0, 16)] = jnp.maximum(lo, 0.0) * a
                    hi = gbuf[r, pl.ds(16, 16)] + gbuf2[r, pl.ds(16, 16)]
                    gbuf[r, pl.ds(16, 16)] = jnp.maximum(hi, 0.0) * a

                for g in range(C // 16):
                    bv = idxv[pl.ds(g * 16, 16)]
                    ok = (bv >= 0) & (bv < 64) & ((lane + g * 16) >= lane_lo)
                    idxv[pl.ds(g * 16, 16)] = jnp.where(ok, bv, 64)
                pltpu.sync_copy(gbuf, acc_s.at[idxv], add=True)
                pltpu.sync_copy(onesb, acc_c.at[idxv], add=True)

        plsc.subcore_barrier()

        @pl.when(sid == 0)
        def _():
            pltpu.sync_copy(acc_s.at[pl.ds(0, 64)], obuf)
            pltpu.sync_copy(obuf, sums_h.at[cid])
            pltpu.sync_copy(acc_c.at[pl.ds(0, 64)], obuf)
            pltpu.sync_copy(obuf, cnts_h.at[cid])

    return pl.kernel(
        body,
        out_type=(jax.ShapeDtypeStruct((NC, 64, 32), F32),
                  jax.ShapeDtypeStruct((NC, 64, 32), F32)),
        mesh=_mesh,
        compiler_params=_sc_params,
        scratch_types=[
            pltpu.VMEM((C,), I32),
            pltpu.VMEM((C, 32), F32),
            pltpu.VMEM((C, 32), F32),
            pltpu.VMEM((C, 32), F32),
            pltpu.VMEM((64, 32), F32),
            pltpu.VMEM_SHARED((ACC, 32), F32),
            pltpu.VMEM_SHARED((ACC, 32), F32),
        ],
    )


def _dense(X, Ws, bs, block_rows=1000, merge_alpha=None):
    split = X.ndim == 3 and X.shape[2] == 16
    pairs = X.ndim == 3 and X.shape[2] == 32
    n = X.shape[1] if (split or pairs) else X.shape[0]
    k = 32 if (split or pairs) else X.shape[1]
    ny = len(Ws)
    W = jnp.stack(Ws)
    b = jnp.stack(bs).reshape(ny, 1, 32)

    def body(x_ref, w_ref, b_ref, *outs):
        if split:
            x = jnp.concatenate([x_ref[0], x_ref[1]], axis=1)
        elif pairs:
            x = jnp.maximum(x_ref[0] + x_ref[1], 0.0) * merge_alpha
        else:
            x = x_ref[...]
        for i in range(ny):
            outs[i][...] = (
                jnp.dot(x, w_ref[i], preferred_element_type=F32) + b_ref[i]
            )

    if split:
        x_spec = pl.BlockSpec((2, block_rows, 16), lambda i: (0, i, 0))
    elif pairs:
        x_spec = pl.BlockSpec((2, block_rows, 32), lambda i: (0, i, 0))
    else:
        x_spec = pl.BlockSpec((block_rows, k), lambda i: (i, 0))
    return pl.pallas_call(
        body,
        grid=(n // block_rows,),
        in_specs=[
            x_spec,
            pl.BlockSpec((ny, k, 32), lambda i: (0, 0, 0)),
            pl.BlockSpec((ny, 1, 32), lambda i: (0, 0, 0)),
        ],
        out_specs=[pl.BlockSpec((block_rows, 32), lambda i: (i, 0))] * ny,
        out_shape=[jax.ShapeDtypeStruct((n, 32), F32)] * ny,
    )(X, W, b)


def _final(s0, c0, s1, c1, s2, c2, Wout, bout):
    def body(s0r, c0r, s1r, c1r, s2r, c2r, wr, br, outr):
        def pooled_planes(sr, cr):
            return jnp.concatenate(
                [sr[0] / jnp.maximum(cr[0], 1.0),
                 sr[1] / jnp.maximum(cr[1], 1.0)], axis=1)

        def pooled_pairs(sr, cr):
            return (sr[0] + sr[1]) / jnp.maximum(cr[0] + cr[1], 1.0)

        cat = jnp.concatenate(
            [pooled_pairs(s0r, c0r), pooled_planes(s1r, c1r),
             pooled_pairs(s2r, c2r)], axis=1)
        z = jnp.dot(cat, wr[...], preferred_element_type=F32) + br[...]
        z = z - jnp.max(z, axis=1, keepdims=True)
        ez = jnp.exp(z)
        outr[...] = ez / jnp.sum(ez, axis=1, keepdims=True)

    return pl.pallas_call(
        body,
        out_shape=jax.ShapeDtypeStruct((64, 32), F32),
    )(s0, c0, s1, c1, s2, c2, Wout, bout.reshape(1, 32))


def _pad_w(W, b):
    kin, kout = W.shape[1], W.shape[2]
    kin_pad = 128 if kin == 128 else 32
    Wp = jnp.zeros((7, kin_pad, 32), F32).at[:, :kin, :kout].set(W)
    bp = jnp.zeros((7, 32), F32).at[:, :kout].set(b)
    return Wp, bp


def _prep_op(rows, cols, vals, full=False):
    e = rows.shape[0]
    ep = _cdiv(e, EPAD) * EPAD
    pad = (0, ep - e)
    r = jnp.pad(rows.astype(I32), pad)
    c = jnp.pad(cols.astype(I32), pad)
    v = lax.bitcast_convert_type(jnp.pad(vals.astype(F32), pad), I32)
    if full:
        ns = ep // C
        return jnp.concatenate(
            [r.reshape(ns, 1, C), c.reshape(ns, 1, C), v.reshape(ns, 1, C)],
            axis=1)
    ns = ep // CSET
    return jnp.concatenate(
        [r.reshape(ns, 2, C), (c * 2).reshape(ns, 2, C),
         v.reshape(ns, 2, C)], axis=1)


def kernel(X0, X1, X2,
           L0_rows, L0_cols, L0_vals,
           L1_rows, L1_cols, L1_vals,
           L2_rows, L2_cols, L2_vals,
           B2D3_rows, B2D3_cols, B2D3_vals,
           D2B1TD1inv_rows, D2B1TD1inv_cols, D2B1TD1inv_vals,
           D1invB1_rows, D1invB1_cols, D1invB1_vals,
           B2TD2inv_rows, B2TD2inv_cols, B2TD2inv_vals,
           batch0, batch1, batch2,
           W1, b1, W2, b2, W3, b3, Wout, bout):
    ii = lambda x: x.astype(I32)
    ff = lambda x: x.astype(F32)

    ops = {
        "L0": _prep_op(L0_rows, L0_cols, L0_vals, full=True),
        "L1": _prep_op(L1_rows, L1_cols, L1_vals),
        "L2": _prep_op(L2_rows, L2_cols, L2_vals, full=True),
        "B2D3": _prep_op(B2D3_rows, B2D3_cols, B2D3_vals),
        "D2B1TD1inv": _prep_op(D2B1TD1inv_rows, D2B1TD1inv_cols,
                               D2B1TD1inv_vals),
        "D1invB1": _prep_op(D1invB1_rows, D1invB1_cols, D1invB1_vals,
                            full=True),
        "B2TD2inv": _prep_op(B2TD2inv_rows, B2TD2inv_cols, B2TD2inv_vals,
                             full=True),
    }

    def layer(x0, x1, x2, W, b):
        Wp, bp = _pad_w(W, b)
        a02 = None if x0.ndim == 2 else 0.5
        y_n2n, y_n2e = _dense(x0, [Wp[0], Wp[1]], [bp[0], bp[1]],
                              merge_alpha=a02)
        y_e2e, y_e2n, y_e2t = _dense(x1, [Wp[2], Wp[3], Wp[4]],
                                     [bp[2], bp[3], bp[4]])
        y_t2e, y_t2t = _dense(x2, [Wp[5], Wp[6]], [bp[5], bp[6]],
                              merge_alpha=a02)

        def run_split(n_out, alpha, pairs):
            counts = tuple(m.shape[0] for m, _ in pairs)
            args = []
            for m, y in pairs:
                args += [m, y.reshape(2 * y.shape[0], 16)]
            return _spmm_acc(n_out, alpha, counts)(*args)

        def run_full(n_out, pairs):
            counts = tuple(m.shape[0] for m, _ in pairs)
            args = []
            for m, y in pairs:
                args += [m, y]
            return _spmm_full(n_out, counts)(*args)

        o0 = run_full(50000, [(ops["L0"], y_n2n), (ops["D1invB1"], y_e2n)])
        o1 = run_split(100000, 1.0 / 3.0, [(ops["L1"], y_e2e),
                                           (ops["D2B1TD1inv"], y_n2e),
                                           (ops["B2D3"], y_t2e)])
        o2 = run_full(50000, [(ops["L2"], y_t2t), (ops["B2TD2inv"], y_e2t)])
        return o0, o1, o2

    x0, x1, x2 = ff(X0), ff(X1), ff(X2)
    x0, x1, x2 = layer(x0, x1, x2, W1, b1)
    x0, x1, x2 = layer(x0, x1, x2, W2, b2)
    x0, x1, x2 = layer(x0, x1, x2, W3, b3)

    s0, c0 = _pool_pairs(50000, 0.5)(x0, ii(batch0))
    s1, c1 = _pool(100000)(x1, ii(batch1))
    s2, c2 = _pool_pairs(50000, 0.5)(x2, ii(batch2))

    return _final(s0, c0, s1, c1, s2, c2, ff(Wout), ff(bout))


# speedup vs baseline: 1.1457x; 1.0105x over previous
import jax
import jax.numpy as jnp
from jax import lax
from jax.experimental import pallas as pl
from jax.experimental.pallas import tpu as pltpu
from jax.experimental.pallas import tpu_sc as plsc

F32 = jnp.float32
I32 = jnp.int32

NC = 2
NS = 16
C = 128
NB = 4
CSET = 2 * C
EPAD = CSET * NS * NB
WC = 200

_mesh = plsc.VectorSubcoreMesh(core_axis_name="c", subcore_axis_name="s")
_sc_params = pltpu.CompilerParams(
    needs_layout_passes=False, use_tc_tiling_on_sc=False)


def _cdiv(a, b):
    return -(-a // b)


def _e16(e):
    return jnp.full((16,), e, I32)


def _spmm_acc(n_out, alpha, set_counts):
    ACC = _cdiv(n_out, C) * C
    nops = len(set_counts)

    def body(*refs):
        ins = refs[: 2 * nops]
        out_h = refs[2 * nops]
        sc = refs[2 * nops + 1:]
        mbuf = sc[0:NB]
        gbuf = sc[NB:2 * NB]
        obuf = sc[2 * NB]
        acc = sc[2 * NB + 1]
        sem_m = sc[2 * NB + 2:2 * NB + 2 + NB]
        sem_g = sc[2 * NB + 2 + NB:2 * NB + 2 + 2 * NB]
        sem_s = sc[2 * NB + 2 + 2 * NB:2 * NB + 2 + 3 * NB]

        cid = lax.axis_index("c")
        sid = lax.axis_index("s")

        zero16 = jnp.zeros((16,), F32)

        @pl.loop(0, C)
        def _zg(r):
            gbuf[0][r, pl.ds(0, 16)] = zero16

        n_zc = ACC // C

        @pl.loop(0, _cdiv(n_zc, NS))
        def _za(j):
            z = sid + j * NS

            @pl.when(z < n_zc)
            def _():
                pltpu.async_copy(gbuf[0].at[pl.ds(0, C)],
                                 acc.at[pl.ds(z * C, C)], sem_m[0])

        @pl.loop(0, _cdiv(n_zc, NS))
        def _zad(j):
            z = sid + j * NS

            @pl.when(z < n_zc)
            def _():
                pltpu.make_async_copy(
                    gbuf[0].at[pl.ds(0, C)],
                    acc.at[pl.ds(0, C)], sem_m[0]).wait()

        plsc.subcore_barrier()

        for oi in range(nops):
            meta_h, y2_h = ins[2 * oi: 2 * oi + 2]
            n_set = set_counts[oi]
            M = n_set // NS

            def issue_meta(k, st, meta_h=meta_h):
                pltpu.async_copy(meta_h.at[st], mbuf[k], sem_m[k])

            def wait_meta(k, meta_h=meta_h):
                pltpu.make_async_copy(meta_h.at[0], mbuf[k], sem_m[k]).wait()

            def xform_gather(k, y2_h=y2_h):
                for h in range(2):
                    @pl.loop(0, C // 16)
                    def _(g, h=h):
                        cv = mbuf[k][2 + h, pl.ds(g * 16, 16)]
                        mbuf[k][2 + h, pl.ds(g * 16, 16)] = cv + cid

                for h in range(2):
                    pltpu.async_copy(
                        y2_h.at[mbuf[k].at[2 + h]],
                        gbuf[k].at[pl.ds(h * C, C)], sem_g[k])

            def wait_gather(k, y2_h=y2_h):
                for h in range(2):
                    pltpu.make_async_copy(
                        y2_h.at[pl.ds(0, C)],
                        gbuf[k].at[pl.ds(h * C, C)], sem_g[k]).wait()

            def scale(k):
                for h in range(2):
                    @pl.loop(0, C // 16)
                    def _(g, h=h):
                        vv = lax.bitcast_convert_type(
                            mbuf[k][4 + h, pl.ds(g * 16, 16)], F32)
                        for e in range(16):
                            sv = vv.at[_e16(e)].get(mode="promise_in_bounds")
                            r = h * C + g * 16 + e
                            gbuf[k][r, pl.ds(0, 16)] = (
                                gbuf[k][r, pl.ds(0, 16)] * sv)

            def issue_scatter(k):
                for h in range(2):
                    pltpu.async_copy(
                        gbuf[k].at[pl.ds(h * C, C)],
                        acc.at[mbuf[k].at[h]], sem_s[k], add=True)

            def wait_scatter(k):
                for h in range(2):
                    pltpu.make_async_copy(
                        gbuf[k].at[pl.ds(h * C, C)],
                        acc.at[pl.ds(0, C)], sem_s[k]).wait()

            def set_of(j):
                return sid + j * NS

            issue_meta(0, set_of(0))
            issue_meta(1, set_of(1))
            wait_meta(0)
            xform_gather(0)

            @pl.loop(0, M, step=NB)
            def _main(j0):
                for t in range(NB):
                    k = t
                    j = j0 + t
                    wait_gather(k)

                    @pl.when(j >= 2)
                    def _():
                        wait_scatter((t + 2) % NB)

                    @pl.when(j + 2 < M)
                    def _():
                        issue_meta((t + 2) % NB, set_of(j + 2))

                    @pl.when(j + 1 < M)
                    def _():
                        wait_meta((t + 1) % NB)
                        xform_gather((t + 1) % NB)

                    scale(k)
                    issue_scatter(k)

            wait_scatter((M - 2) % NB)
            wait_scatter((M - 1) % NB)

        plsc.subcore_barrier()

        n_wc = n_out // WC
        a = jnp.float32(alpha)

        @pl.loop(0, _cdiv(n_wc, NS))
        def _wb(j):
            w = sid + j * NS

            @pl.when(w < n_wc)
            def _():
                pltpu.sync_copy(acc.at[pl.ds(w * WC, WC)], obuf)

                @pl.loop(0, WC)
                def _r(r):
                    lo = obuf[r, pl.ds(0, 16)]
                    obuf[r, pl.ds(0, 16)] = jnp.maximum(lo, 0.0) * a

                pltpu.sync_copy(obuf, out_h.at[cid, pl.ds(w * WC, WC)])

    return pl.kernel(
        body,
        out_type=jax.ShapeDtypeStruct((NC, n_out, 16), F32),
        mesh=_mesh,
        compiler_params=_sc_params,
        scratch_types=(
            [pltpu.VMEM((6, C), I32) for _ in range(NB)]
            + [pltpu.VMEM((CSET, 16), F32) for _ in range(NB)]
            + [pltpu.VMEM((WC, 16), F32)]
            + [pltpu.VMEM_SHARED((ACC, 16), F32)]
            + [pltpu.SemaphoreType.DMA for _ in range(3 * NB)]
        ),
    )


def _spmm_full(n_out, set_counts):
    ACC = _cdiv(n_out, C) * C
    nops = len(set_counts)
    NW = NC * NS

    def body(*refs):
        ins = refs[: 2 * nops]
        out_h = refs[2 * nops]
        sc = refs[2 * nops + 1:]
        mbuf = sc[0:NB]
        gbuf = sc[NB:2 * NB]
        acc = sc[2 * NB]
        sem_m = sc[2 * NB + 1:2 * NB + 1 + NB]
        sem_g = sc[2 * NB + 1 + NB:2 * NB + 1 + 2 * NB]
        sem_s = sc[2 * NB + 1 + 2 * NB:2 * NB + 1 + 3 * NB]

        cid = lax.axis_index("c")
        sid = lax.axis_index("s")
        wid = sid * NC + cid

        zero16 = jnp.zeros((16,), F32)

        @pl.loop(0, C)
        def _zg(r):
            gbuf[0][r, pl.ds(0, 16)] = zero16
            gbuf[0][r, pl.ds(16, 16)] = zero16

        n_zc = ACC // C

        @pl.loop(0, _cdiv(n_zc, NS))
        def _za(j):
            z = sid + j * NS

            @pl.when(z < n_zc)
            def _():
                pltpu.async_copy(gbuf[0], acc.at[pl.ds(z * C, C)], sem_m[0])

        @pl.loop(0, _cdiv(n_zc, NS))
        def _zad(j):
            z = sid + j * NS

            @pl.when(z < n_zc)
            def _():
                pltpu.make_async_copy(
                    gbuf[0], acc.at[pl.ds(0, C)], sem_m[0]).wait()

        plsc.subcore_barrier()

        for oi in range(nops):
            meta_h, y_h = ins[2 * oi: 2 * oi + 2]
            n_set = set_counts[oi]
            M = n_set // NW

            def issue_meta(k, st, meta_h=meta_h):
                pltpu.async_copy(meta_h.at[st], mbuf[k], sem_m[k])

            def wait_meta(k, meta_h=meta_h):
                pltpu.make_async_copy(meta_h.at[0], mbuf[k], sem_m[k]).wait()

            def issue_gather(k, y_h=y_h):
                pltpu.async_copy(
                    y_h.at[mbuf[k].at[1]], gbuf[k], sem_g[k])

            def wait_gather(k, y_h=y_h):
                pltpu.make_async_copy(
                    y_h.at[pl.ds(0, C)], gbuf[k], sem_g[k]).wait()

            def scale(k):
                @pl.loop(0, C // 16)
                def _(g):
                    vv = lax.bitcast_convert_type(
                        mbuf[k][2, pl.ds(g * 16, 16)], F32)
                    for e in range(16):
                        sv = vv.at[_e16(e)].get(mode="promise_in_bounds")
                        r = g * 16 + e
                        gbuf[k][r, pl.ds(0, 16)] = (
                            gbuf[k][r, pl.ds(0, 16)] * sv)
                        gbuf[k][r, pl.ds(16, 16)] = (
                            gbuf[k][r, pl.ds(16, 16)] * sv)

            def issue_scatter(k):
                pltpu.async_copy(
                    gbuf[k], acc.at[mbuf[k].at[0]], sem_s[k], add=True)

            def wait_scatter(k):
                pltpu.make_async_copy(
                    gbuf[k], acc.at[pl.ds(0, C)], sem_s[k]).wait()

            def set_of(j):
                return wid + j * NW

            issue_meta(0, set_of(0))
            issue_meta(1, set_of(1))
            wait_meta(0)
            issue_gather(0)

            @pl.loop(0, M, step=NB)
            def _main(j0):
                for t in range(NB):
                    k = t
                    j = j0 + t
                    wait_gather(k)

                    @pl.when(j >= 2)
                    def _():
                        wait_scatter((t + 2) % NB)

                    @pl.when(j + 2 < M)
                    def _():
                        issue_meta((t + 2) % NB, set_of(j + 2))

                    @pl.when(j + 1 < M)
                    def _():
                        wait_meta((t + 1) % NB)
                        issue_gather((t + 1) % NB)

                    scale(k)
                    issue_scatter(k)

            wait_scatter((M - 2) % NB)
            wait_scatter((M - 1) % NB)

        plsc.subcore_barrier()

        n_wc = _cdiv(n_out, C)

        @pl.loop(0, _cdiv(n_wc, NS))
        def _wb(j):
            w = sid + j * NS

            @pl.when(w < n_wc)
            def _():
                base = jnp.minimum(w * C, n_out - C)
                pltpu.async_copy(acc.at[pl.ds(base, C)],
                                 out_h.at[cid, pl.ds(base, C)], sem_m[0])

        @pl.loop(0, _cdiv(n_wc, NS))
        def _wbd(j):
            w = sid + j * NS

            @pl.when(w < n_wc)
            def _():
                pltpu.make_async_copy(
                    acc.at[pl.ds(0, C)],
                    out_h.at[cid, pl.ds(0, C)], sem_m[0]).wait()

    return pl.kernel(
        body,
        out_type=jax.ShapeDtypeStruct((NC, n_out, 32), F32),
        mesh=_mesh,
        compiler_params=_sc_params,
        scratch_types=(
            [pltpu.VMEM((3, C), I32) for _ in range(NB)]
            + [pltpu.VMEM((C, 32), F32) for _ in range(NB)]
            + [pltpu.VMEM_SHARED((ACC, 32), F32)]
            + [pltpu.SemaphoreType.DMA for _ in range(3 * NB)]
        ),
    )


def _pool(n_rows):
    ACC = 128

    def body(x_h, b_h, sums_h, cnts_h, idxv, gbuf, onesb, obuf, acc_s, acc_c):
        cid = lax.axis_index("c")
        sid = lax.axis_index("s")

        zero16 = jnp.zeros((16,), F32)
        one16 = jnp.ones((16,), F32)

        @pl.loop(0, C)
        def _init(r):
            gbuf[r, pl.ds(0, 16)] = zero16
            onesb[r, pl.ds(0, 16)] = one16

        @pl.when(sid == 0)
        def _():
            pltpu.sync_copy(gbuf, acc_s)
            pltpu.sync_copy(gbuf, acc_c)

        plsc.subcore_barrier()

        lane = lax.iota(I32, 16)
        n_ch = _cdiv(n_rows, C)

        @pl.loop(0, _cdiv(n_ch, NS))
        def _rows(j):
            ch = sid + j * NS

            @pl.when(ch < n_ch)
            def _():
                base0 = ch * C
                base = jnp.minimum(base0, n_rows - C)
                lane_lo = base0 - base
                pltpu.sync_copy(x_h.at[cid, pl.ds(base, C)], gbuf)
                pltpu.sync_copy(b_h.at[pl.ds(base, C)], idxv)
                for g in range(C // 16):
                    bv = idxv[pl.ds(g * 16, 16)]
                    ok = (bv >= 0) & (bv < 64) & ((lane + g * 16) >= lane_lo)
                    idxv[pl.ds(g * 16, 16)] = jnp.where(ok, bv, 64)
                pltpu.sync_copy(gbuf, acc_s.at[idxv], add=True)
                pltpu.sync_copy(onesb, acc_c.at[idxv], add=True)

        plsc.subcore_barrier()

        @pl.when(sid == 0)
        def _():
            pltpu.sync_copy(acc_s.at[pl.ds(0, 64)], obuf)
            pltpu.sync_copy(obuf, sums_h.at[cid])
            pltpu.sync_copy(acc_c.at[pl.ds(0, 64)], obuf)
            pltpu.sync_copy(obuf, cnts_h.at[cid])

    return pl.kernel(
        body,
        out_type=(jax.ShapeDtypeStruct((NC, 64, 16), F32),
                  jax.ShapeDtypeStruct((NC, 64, 16), F32)),
        mesh=_mesh,
        compiler_params=_sc_params,
        scratch_types=[
            pltpu.VMEM((C,), I32),
            pltpu.VMEM((C, 16), F32),
            pltpu.VMEM((C, 16), F32),
            pltpu.VMEM((64, 16), F32),
            pltpu.VMEM_SHARED((ACC, 16), F32),
            pltpu.VMEM_SHARED((ACC, 16), F32),
        ],
    )


def _pool_pairs(n_rows, alpha):
    ACC = 128

    def body(x_h, b_h, sums_h, cnts_h, idxv, gbuf, gbuf2, onesb, obuf,
             acc_s, acc_c):
        cid = lax.axis_index("c")
        sid = lax.axis_index("s")
        wid = sid * NC + cid

        zero16 = jnp.zeros((16,), F32)
        one16 = jnp.ones((16,), F32)
        a = jnp.float32(alpha)

        @pl.loop(0, C)
        def _init(r):
            gbuf[r, pl.ds(0, 16)] = zero16
            gbuf[r, pl.ds(16, 16)] = zero16
            onesb[r, pl.ds(0, 16)] = one16
            onesb[r, pl.ds(16, 16)] = one16

        @pl.when(sid == 0)
        def _():
            pltpu.sync_copy(gbuf, acc_s)
            pltpu.sync_copy(gbuf, acc_c)

        plsc.subcore_barrier()

        lane = lax.iota(I32, 16)
        n_ch = _cdiv(n_rows, C)

        @pl.loop(0, _cdiv(n_ch, NC * NS))
        def _rows(j):
            ch = wid + j * NC * NS

            @pl.when(ch < n_ch)
            def _():
                base0 = ch * C
                base = jnp.minimum(base0, n_rows - C)
                lane_lo = base0 - base
                pltpu.sync_copy(x_h.at[0, pl.ds(base, C)], gbuf)
                pltpu.sync_copy(x_h.at[1, pl.ds(base, C)], gbuf2)
                pltpu.sync_copy(b_h.at[pl.ds(base, C)], idxv)

                @pl.loop(0, C)
                def _m(r):
                    lo = gbuf[r, pl.ds(0, 16)] + gbuf2[r, pl.ds(0, 16)]
                    gbuf[r, pl.ds(0, 16)] = jnp.maximum(lo, 0.0) * a
                    hi = gbuf[r, pl.ds(16, 16)] + gbuf2[r, pl.ds(16, 16)]
                    gbuf[r, pl.ds(16, 16)] = jnp.maximum(hi, 0.0) * a

                for g in range(C // 16):
                    bv = idxv[pl.ds(g * 16, 16)]
                    ok = (bv >= 0) & (bv < 64) & ((lane + g * 16) >= lane_lo)
                    idxv[pl.ds(g * 16, 16)] = jnp.where(ok, bv, 64)
                pltpu.sync_copy(gbuf, acc_s.at[idxv], add=True)
                pltpu.sync_copy(onesb, acc_c.at[idxv], add=True)

        plsc.subcore_barrier()

        @pl.when(sid == 0)
        def _():
            pltpu.sync_copy(acc_s.at[pl.ds(0, 64)], obuf)
            pltpu.sync_copy(obuf, sums_h.at[cid])
            pltpu.sync_copy(acc_c.at[pl.ds(0, 64)], obuf)
            pltpu.sync_copy(obuf, cnts_h.at[cid])

    return pl.kernel(
        body,
        out_type=(jax.ShapeDtypeStruct((NC, 64, 32), F32),
                  jax.ShapeDtypeStruct((NC, 64, 32), F32)),
        mesh=_mesh,
        compiler_params=_sc_params,
        scratch_types=[
            pltpu.VMEM((C,), I32),
            pltpu.VMEM((C, 32), F32),
            pltpu.VMEM((C, 32), F32),
            pltpu.VMEM((C, 32), F32),
            pltpu.VMEM((64, 32), F32),
            pltpu.VMEM_SHARED((ACC, 32), F32),
            pltpu.VMEM_SHARED((ACC, 32), F32),
        ],
    )


def _dense(X, Ws, bs, block_rows=1000, merge_alpha=None):
    split = X.ndim == 3 and X.shape[2] == 16
    pairs = X.ndim == 3 and X.shape[2] == 32
    n = X.shape[1] if (split or pairs) else X.shape[0]
    k = 32 if (split or pairs) else X.shape[1]
    ny = len(Ws)
    W = jnp.stack(Ws)
    b = jnp.stack(bs).reshape(ny, 1, 32)

    def body(x_ref, w_ref, b_ref, *outs):
        if split:
            x = jnp.concatenate([x_ref[0], x_ref[1]], axis=1)
        elif pairs:
            x = jnp.maximum(x_ref[0] + x_ref[1], 0.0) * merge_alpha
        else:
            x = x_ref[...]
        for i in range(ny):
            outs[i][...] = (
                jnp.dot(x, w_ref[i], preferred_element_type=F32) + b_ref[i]
            )

    if split:
        x_spec = pl.BlockSpec((2, block_rows, 16), lambda i: (0, i, 0))
    elif pairs:
        x_spec = pl.BlockSpec((2, block_rows, 32), lambda i: (0, i, 0))
    else:
        x_spec = pl.BlockSpec((block_rows, k), lambda i: (i, 0))
    return pl.pallas_call(
        body,
        grid=(n // block_rows,),
        in_specs=[
            x_spec,
            pl.BlockSpec((ny, k, 32), lambda i: (0, 0, 0)),
            pl.BlockSpec((ny, 1, 32), lambda i: (0, 0, 0)),
        ],
        out_specs=[pl.BlockSpec((block_rows, 32), lambda i: (i, 0))] * ny,
        out_shape=[jax.ShapeDtypeStruct((n, 32), F32)] * ny,
    )(X, W, b)


def _final(s0, c0, s1, c1, s2, c2, Wout, bout):
    def body(s0r, c0r, s1r, c1r, s2r, c2r, wr, br, outr):
        def pooled_planes(sr, cr):
            return jnp.concatenate(
                [sr[0] / jnp.maximum(cr[0], 1.0),
                 sr[1] / jnp.maximum(cr[1], 1.0)], axis=1)

        def pooled_pairs(sr, cr):
            return (sr[0] + sr[1]) / jnp.maximum(cr[0] + cr[1], 1.0)

        cat = jnp.concatenate(
            [pooled_pairs(s0r, c0r), pooled_planes(s1r, c1r),
             pooled_pairs(s2r, c2r)], axis=1)
        z = jnp.dot(cat, wr[...], preferred_element_type=F32) + br[...]
        z = z - jnp.max(z, axis=1, keepdims=True)
        ez = jnp.exp(z)
        outr[...] = ez / jnp.sum(ez, axis=1, keepdims=True)

    return pl.pallas_call(
        body,
        out_shape=jax.ShapeDtypeStruct((64, 32), F32),
    )(s0, c0, s1, c1, s2, c2, Wout, bout.reshape(1, 32))


def _pad_w(W, b):
    kin, kout = W.shape[1], W.shape[2]
    kin_pad = 128 if kin == 128 else 32
    Wp = jnp.zeros((7, kin_pad, 32), F32).at[:, :kin, :kout].set(W)
    bp = jnp.zeros((7, 32), F32).at[:, :kout].set(b)
    return Wp, bp


def _prep_op(rows, cols, vals, full=False):
    e = rows.shape[0]
    ep = _cdiv(e, EPAD) * EPAD
    pad = (0, ep - e)
    r = jnp.pad(rows.astype(I32), pad)
    c = jnp.pad(cols.astype(I32), pad)
    v = lax.bitcast_convert_type(jnp.pad(vals.astype(F32), pad), I32)
    if full:
        ns = ep // C
        return jnp.concatenate(
            [r.reshape(ns, 1, C), c.reshape(ns, 1, C), v.reshape(ns, 1, C)],
            axis=1)
    ns = ep // CSET
    return jnp.concatenate(
        [r.reshape(ns, 2, C), (c * 2).reshape(ns, 2, C),
         v.reshape(ns, 2, C)], axis=1)


def kernel(X0, X1, X2,
           L0_rows, L0_cols, L0_vals,
           L1_rows, L1_cols, L1_vals,
           L2_rows, L2_cols, L2_vals,
           B2D3_rows, B2D3_cols, B2D3_vals,
           D2B1TD1inv_rows, D2B1TD1inv_cols, D2B1TD1inv_vals,
           D1invB1_rows, D1invB1_cols, D1invB1_vals,
           B2TD2inv_rows, B2TD2inv_cols, B2TD2inv_vals,
           batch0, batch1, batch2,
           W1, b1, W2, b2, W3, b3, Wout, bout):
    ii = lambda x: x.astype(I32)
    ff = lambda x: x.astype(F32)

    ops = {
        "L0": _prep_op(L0_rows, L0_cols, L0_vals, full=True),
        "L1": _prep_op(L1_rows, L1_cols, L1_vals),
        "L2": _prep_op(L2_rows, L2_cols, L2_vals, full=True),
        "B2D3": _prep_op(B2D3_rows, B2D3_cols, B2D3_vals),
        "D2B1TD1inv": _prep_op(D2B1TD1inv_rows, D2B1TD1inv_cols,
                               D2B1TD1inv_vals),
        "D1invB1": _prep_op(D1invB1_rows, D1invB1_cols, D1invB1_vals,
                            full=True),
        "B2TD2inv": _prep_op(B2TD2inv_rows, B2TD2inv_cols, B2TD2inv_vals,
                             full=True),
    }

    def layer(x0, x1, x2, W, b):
        Wp, bp = _pad_w(W, b)
        a02 = None if x0.ndim == 2 else 0.5
        y_n2n, y_n2e = _dense(x0, [Wp[0], Wp[1]], [bp[0], bp[1]],
                              merge_alpha=a02)
        y_e2e, y_e2n, y_e2t = _dense(x1, [Wp[2], Wp[3], Wp[4]],
                                     [bp[2], bp[3], bp[4]])
        y_t2e, y_t2t = _dense(x2, [Wp[5], Wp[6]], [bp[5], bp[6]],
                              merge_alpha=a02)

        def run_split(n_out, alpha, pairs):
            counts = tuple(m.shape[0] for m, _ in pairs)
            args = []
            for m, y in pairs:
                args += [m, y.reshape(2 * y.shape[0], 16)]
            return _spmm_acc(n_out, alpha, counts)(*args)

        def run_full(n_out, pairs):
            counts = tuple(m.shape[0] for m, _ in pairs)
            args = []
            for m, y in pairs:
                args += [m, y]
            return _spmm_full(n_out, counts)(*args)

        o0 = run_full(50000, [(ops["L0"], y_n2n), (ops["D1invB1"], y_e2n)])
        o1 = run_split(100000, 1.0 / 3.0, [(ops["L1"], y_e2e),
                                           (ops["D2B1TD1inv"], y_n2e),
                                           (ops["B2D3"], y_t2e)])
        o2 = run_full(50000, [(ops["L2"], y_t2t), (ops["B2TD2inv"], y_e2t)])
        return o0, o1, o2

    x0, x1, x2 = ff(X0), ff(X1), ff(X2)
    x0, x1, x2 = layer(x0, x1, x2, W1, b1)
    x0, x1, x2 = layer(x0, x1, x2, W2, b2)
    x0, x1, x2 = layer(x0, x1, x2, W3, b3)

    s0, c0 = _pool_pairs(50000, 0.5)(x0, ii(batch0))
    s1, c1 = _pool(100000)(x1, ii(batch1))
    s2, c2 = _pool_pairs(50000, 0.5)(x2, ii(batch2))

    return _final(s0, c0, s1, c1, s2, c2, ff(Wout), ff(bout))
